# baseline (speedup 1.0000x reference)
import functools

import jax
import jax.numpy as jnp
from jax import lax
from jax.experimental import pallas as pl
from jax.experimental.pallas import tpu as pltpu
from jax.experimental.pallas import tpu_sc as plsc

N = 10000
D = 128
E = 320000
NC = 2
NS = 16
NW = NC * NS
CHUNK = 128
NCH = 80
HNCH = NCH // 2
EPT = NCH * CHUNK
E_PAD = NW * EPT
DCH = 128
DNCH = 80
N_PAD = 10240
RPT = N_PAD // NS

_MESH = plsc.VectorSubcoreMesh(core_axis_name="c", subcore_axis_name="s")


@functools.partial(
    pl.kernel,
    out_type=jax.ShapeDtypeStruct((NC, N_PAD, 16), jnp.float32),
    mesh=_MESH,
    scratch_types=[
        pltpu.VMEM((DNCH, DCH), jnp.int32),
        pltpu.VMEM((DCH, 16), jnp.float32),
        pltpu.VMEM((DCH, 16), jnp.float32),
        pltpu.VMEM_SHARED((N_PAD, 16), jnp.float32),
    ],
)
def _sc_degree(dst_hbm, out_hbm, idx_v, ones_v, zero_v, acc_sh):
    c = lax.axis_index("c")
    s = lax.axis_index("s")
    wid = c * NS + s
    pltpu.sync_copy(dst_hbm.at[wid], idx_v)

    def _fill(i, _):
        ones_v[i] = jnp.ones((16,), jnp.float32)
        zero_v[i] = jnp.zeros((16,), jnp.float32)
        return 0

    lax.fori_loop(0, DCH, _fill, 0)

    def _zero(z, _):
        pltpu.sync_copy(zero_v, acc_sh.at[pl.ds(s * RPT + z * DCH, DCH)])
        return 0

    lax.fori_loop(0, RPT // DCH, _zero, 0)
    plsc.subcore_barrier()

    def _count(j, _):
        pltpu.sync_copy(ones_v, acc_sh.at[idx_v.at[j]], add=True)
        return 0

    lax.fori_loop(0, DNCH, _count, 0)
    plsc.subcore_barrier()
    pltpu.sync_copy(acc_sh.at[pl.ds(s * RPT, RPT)],
                    out_hbm.at[c, pl.ds(s * RPT, RPT)])


@functools.partial(
    pl.kernel,
    out_type=jax.ShapeDtypeStruct((NC, N_PAD, D), jnp.float32),
    mesh=_MESH,
    scratch_types=[
        pltpu.VMEM((NCH, CHUNK), jnp.int32),
        pltpu.VMEM((NCH, CHUNK), jnp.int32),
        pltpu.VMEM((CHUNK, D), jnp.float32),
        pltpu.VMEM_SHARED((N_PAD, D), jnp.float32),
        pltpu.SemaphoreType.DMA,
    ],
)
def _sc_edge_pass(y_hbm, src_hbm, dst_hbm, out_hbm, sidx, didx, rows, acc_sh,
                  gsem):
    c = lax.axis_index("c")
    sub = lax.axis_index("s")
    wid = c * NS + sub
    pltpu.sync_copy(src_hbm.at[wid], sidx)
    pltpu.sync_copy(dst_hbm.at[wid], didx)

    def _zrow(i, _):
        def _zcol(k, _):
            rows[i, pl.ds(k * 16, 16)] = jnp.zeros((16,), jnp.float32)
            return 0
        lax.fori_loop(0, D // 16, _zcol, 0)
        return 0

    lax.fori_loop(0, CHUNK, _zrow, 0)

    def _zero(z, _):
        pltpu.sync_copy(rows, acc_sh.at[pl.ds(sub * RPT + z * CHUNK, CHUNK)])
        return 0

    lax.fori_loop(0, RPT // CHUNK, _zero, 0)
    plsc.subcore_barrier()

    def _edge_chunk(j, _):
        pltpu.async_copy(y_hbm.at[sidx.at[j]], rows, gsem).wait()
        pltpu.sync_copy(rows, acc_sh.at[didx.at[j]], add=True)
        return 0

    lax.fori_loop(0, NCH, _edge_chunk, 0)
    plsc.subcore_barrier()
    pltpu.sync_copy(acc_sh.at[pl.ds(sub * RPT, RPT)],
                    out_hbm.at[c, pl.ds(sub * RPT, RPT)])


_R = 1024


def _dinv_of(deg_ref):
    d16 = deg_ref[...]
    return lax.rsqrt(1.0 + d16[0, :, 0] + d16[1, :, 0])[:, None]


def _tc_pre_body(deg_ref, x_ref, w_ref, y_ref):
    y_ref[...] = _dinv_of(deg_ref) * jnp.dot(
        x_ref[...], w_ref[...], preferred_element_type=jnp.float32)


def _tc_mid_body(deg_ref, acc_ref, y_ref, b_ref, w_ref, out_ref):
    dinv = _dinv_of(deg_ref)
    x2 = jnp.maximum(
        dinv * (acc_ref[0] + acc_ref[1] + y_ref[...]) + b_ref[...], 0.0)
    out_ref[...] = dinv * jnp.dot(
        x2, w_ref[...], preferred_element_type=jnp.float32)


def _tc_fin_body(deg_ref, acc_ref, y_ref, b_ref, out_ref):
    dinv = _dinv_of(deg_ref)
    out_ref[...] = dinv * (acc_ref[0] + acc_ref[1] + y_ref[...]) + b_ref[...]


_DEG_SPEC = pl.BlockSpec((NC, _R, 16), lambda i: (0, i, 0))
_ACC_SPEC = pl.BlockSpec((NC, _R, D), lambda i: (0, i, 0))
_ROW_SPEC = pl.BlockSpec((_R, D), lambda i: (i, 0))
_W_SPEC = pl.BlockSpec((D, D), lambda i: (0, 0))
_B_SPEC = pl.BlockSpec((1, D), lambda i: (0, 0))
_OUT_TYPE = jax.ShapeDtypeStruct((N_PAD, D), jnp.float32)
_GRID = (N_PAD // _R,)


def _tc_pre(deg16, x_pad, W):
    return pl.pallas_call(
        _tc_pre_body, grid=_GRID,
        in_specs=[_DEG_SPEC, _ROW_SPEC, _W_SPEC],
        out_specs=_ROW_SPEC, out_shape=_OUT_TYPE,
    )(deg16, x_pad, W)


def _tc_mid(deg16, acc, y, b_row, W):
    return pl.pallas_call(
        _tc_mid_body, grid=_GRID,
        in_specs=[_DEG_SPEC, _ACC_SPEC, _ROW_SPEC, _B_SPEC, _W_SPEC],
        out_specs=_ROW_SPEC, out_shape=_OUT_TYPE,
    )(deg16, acc, y, b_row, W)


def _tc_fin(deg16, acc, y, b_row):
    return pl.pallas_call(
        _tc_fin_body, grid=_GRID,
        in_specs=[_DEG_SPEC, _ACC_SPEC, _ROW_SPEC, _B_SPEC],
        out_specs=_ROW_SPEC, out_shape=_OUT_TYPE,
    )(deg16, acc, y, b_row)


def kernel(edge_index, emb, W1, b1, W2, b2):
    src = edge_index[0]
    dst = edge_index[1]
    pad = jnp.full((E_PAD - E,), N, jnp.int32)
    src_flat = jnp.concatenate([src, pad])
    dst_flat = jnp.concatenate([dst, pad])
    srcp = src_flat.reshape(NW, NCH, CHUNK)
    dstp = dst_flat.reshape(NW, NCH, CHUNK)
    dst_deg = dst_flat.reshape(NW, DNCH, DCH)
    emb_pad = jnp.pad(emb, ((0, N_PAD - N), (0, 0)))
    b1r = b1.reshape(1, D)
    b2r = b2.reshape(1, D)

    deg16 = _sc_degree(dst_deg)
    y1 = _tc_pre(deg16, emb_pad, W1)
    acc1 = _sc_edge_pass(y1, srcp, dstp)
    y2 = _tc_mid(deg16, acc1, y1, b1r, W2)
    acc2 = _sc_edge_pass(y2, srcp, dstp)
    out_pad = _tc_fin(deg16, acc2, y2, b2r)
    return out_pad[:N]


# speedup vs baseline: 1.5664x; 1.5664x over previous
import functools

import jax
import jax.numpy as jnp
from jax import lax
from jax.experimental import pallas as pl
from jax.experimental.pallas import tpu as pltpu
from jax.experimental.pallas import tpu_sc as plsc

N = 10000
D = 128
E = 320000
NC = 2
NS = 16
NW = NC * NS
CHUNK = 128
NCH = 79
EPT = NCH * CHUNK
E_PAD = NW * EPT
N_PAD = 10240
RPT = N_PAD // NS

_MESH = plsc.VectorSubcoreMesh(core_axis_name="c", subcore_axis_name="s")


@functools.partial(
    pl.kernel,
    out_type=jax.ShapeDtypeStruct((NC, N_PAD, D), jnp.float32),
    mesh=_MESH,
    scratch_types=[
        pltpu.VMEM((NCH, CHUNK), jnp.int32),
        pltpu.VMEM((CHUNK, D), jnp.float32),
        pltpu.VMEM_SHARED((N_PAD, D), jnp.float32),
    ],
)
def _sc_degree(dst_hbm, zeros_hbm, ones_hbm, out_hbm, idx_v, rows, acc_sh):
    c = lax.axis_index("c")
    s = lax.axis_index("s")
    wid = c * NS + s
    pltpu.sync_copy(dst_hbm.at[wid], idx_v)
    pltpu.sync_copy(zeros_hbm, rows)

    def _zero(z, _):
        pltpu.sync_copy(rows, acc_sh.at[pl.ds(s * RPT + z * CHUNK, CHUNK)])
        return 0

    lax.fori_loop(0, RPT // CHUNK, _zero, 0)
    pltpu.sync_copy(ones_hbm, rows)
    plsc.subcore_barrier()

    def _count(j, _):
        pltpu.sync_copy(rows, acc_sh.at[idx_v.at[j]], add=True)
        return 0

    lax.fori_loop(0, NCH, _count, 0)
    plsc.subcore_barrier()
    pltpu.sync_copy(acc_sh.at[pl.ds(s * RPT, RPT)],
                    out_hbm.at[c, pl.ds(s * RPT, RPT)])


@functools.partial(
    pl.kernel,
    out_type=jax.ShapeDtypeStruct((NC, N_PAD, D), jnp.float32),
    mesh=_MESH,
    scratch_types=[
        pltpu.VMEM((NCH, CHUNK), jnp.int32),
        pltpu.VMEM((NCH, CHUNK), jnp.int32),
        pltpu.VMEM((CHUNK, D), jnp.float32),
        pltpu.VMEM_SHARED((N_PAD, D), jnp.float32),
        pltpu.SemaphoreType.DMA,
    ],
)
def _sc_edge_pass(y_hbm, src_hbm, dst_hbm, zeros_hbm, out_hbm,
                  sidx, didx, rows, acc_sh, gsem):
    c = lax.axis_index("c")
    s = lax.axis_index("s")
    wid = c * NS + s
    pltpu.sync_copy(src_hbm.at[wid], sidx)
    pltpu.sync_copy(dst_hbm.at[wid], didx)
    pltpu.sync_copy(zeros_hbm, rows)

    def _zero(z, _):
        pltpu.sync_copy(rows, acc_sh.at[pl.ds(s * RPT + z * CHUNK, CHUNK)])
        return 0

    lax.fori_loop(0, RPT // CHUNK, _zero, 0)
    plsc.subcore_barrier()

    def _edge_chunk(j, _):
        pltpu.async_copy(y_hbm.at[sidx.at[j]], rows, gsem).wait()
        pltpu.sync_copy(rows, acc_sh.at[didx.at[j]], add=True)
        return 0

    lax.fori_loop(0, NCH, _edge_chunk, 0)
    plsc.subcore_barrier()
    pltpu.sync_copy(acc_sh.at[pl.ds(s * RPT, RPT)],
                    out_hbm.at[c, pl.ds(s * RPT, RPT)])


_R = 1024


def _dinv_of(deg_ref):
    d16 = deg_ref[...]
    return lax.rsqrt(1.0 + d16[0, :, 0] + d16[1, :, 0])[:, None]


def _tc_pre_body(deg_ref, x_ref, w_ref, y_ref):
    y_ref[...] = _dinv_of(deg_ref) * jnp.dot(
        x_ref[...], w_ref[...], preferred_element_type=jnp.float32)


def _tc_mid_body(deg_ref, acc_ref, y_ref, b_ref, w_ref, out_ref):
    dinv = _dinv_of(deg_ref)
    x2 = jnp.maximum(
        dinv * (acc_ref[0] + acc_ref[1] + y_ref[...]) + b_ref[...], 0.0)
    out_ref[...] = dinv * jnp.dot(
        x2, w_ref[...], preferred_element_type=jnp.float32)


def _tc_fin_body(deg_ref, acc_ref, y_ref, b_ref, out_ref):
    dinv = _dinv_of(deg_ref)
    out_ref[...] = dinv * (acc_ref[0] + acc_ref[1] + y_ref[...]) + b_ref[...]


_DEG_SPEC = pl.BlockSpec((NC, _R, D), lambda i: (0, i, 0))
_ACC_SPEC = pl.BlockSpec((NC, _R, D), lambda i: (0, i, 0))
_ROW_SPEC = pl.BlockSpec((_R, D), lambda i: (i, 0))
_W_SPEC = pl.BlockSpec((D, D), lambda i: (0, 0))
_B_SPEC = pl.BlockSpec((1, D), lambda i: (0, 0))
_OUT_TYPE = jax.ShapeDtypeStruct((N_PAD, D), jnp.float32)
_GRID = (N_PAD // _R,)


def _tc_pre(deg16, x_pad, W):
    return pl.pallas_call(
        _tc_pre_body, grid=_GRID,
        in_specs=[_DEG_SPEC, _ROW_SPEC, _W_SPEC],
        out_specs=_ROW_SPEC, out_shape=_OUT_TYPE,
    )(deg16, x_pad, W)


def _tc_mid(deg16, acc, y, b_row, W):
    return pl.pallas_call(
        _tc_mid_body, grid=_GRID,
        in_specs=[_DEG_SPEC, _ACC_SPEC, _ROW_SPEC, _B_SPEC, _W_SPEC],
        out_specs=_ROW_SPEC, out_shape=_OUT_TYPE,
    )(deg16, acc, y, b_row, W)


def _tc_fin(deg16, acc, y, b_row):
    return pl.pallas_call(
        _tc_fin_body, grid=_GRID,
        in_specs=[_DEG_SPEC, _ACC_SPEC, _ROW_SPEC, _B_SPEC],
        out_specs=_ROW_SPEC, out_shape=_OUT_TYPE,
    )(deg16, acc, y, b_row)


def kernel(edge_index, emb, W1, b1, W2, b2):
    src = edge_index[0]
    dst = edge_index[1]
    pad = jnp.full((E_PAD - E,), N, jnp.int32)
    srcp = jnp.concatenate([src, pad]).reshape(NW, NCH, CHUNK)
    dstp = jnp.concatenate([dst, pad]).reshape(NW, NCH, CHUNK)
    emb_pad = jnp.pad(emb, ((0, N_PAD - N), (0, 0)))
    b1r = b1.reshape(1, D)
    b2r = b2.reshape(1, D)

    zrows = jnp.zeros((CHUNK, D), jnp.float32)
    orows = jnp.ones((CHUNK, D), jnp.float32)

    deg16 = _sc_degree(dstp, zrows, orows)
    y1 = _tc_pre(deg16, emb_pad, W1)
    acc1 = _sc_edge_pass(y1, srcp, dstp, zrows)
    y2 = _tc_mid(deg16, acc1, y1, b1r, W2)
    acc2 = _sc_edge_pass(y2, srcp, dstp, zrows)
    out_pad = _tc_fin(deg16, acc2, y2, b2r)
    return out_pad[:N]
